# baseline (speedup 1.0000x reference)
import functools

import jax
import jax.numpy as jnp
from jax import lax
from jax.experimental import pallas as pl
from jax.experimental.pallas import tpu as pltpu
from jax.experimental.pallas import tpu_sc as plsc

_N = 50000
_K = 2048
_NS = 16
_CHUNK = 3136
_NCH = _CHUNK // 16
_NPAD = _NS * _CHUNK


@functools.cache
def _build_fps_sc():
  mesh = plsc.VectorSubcoreMesh(
      core_axis_name="c", subcore_axis_name="s", num_cores=1, num_subcores=_NS
  )

  @functools.partial(
      pl.kernel,
      out_type=jax.ShapeDtypeStruct((_K * 3,), jnp.float32),
      mesh=mesh,
      compiler_params=pltpu.CompilerParams(needs_layout_passes=False),
      scratch_types=[
          pltpu.VMEM((_CHUNK,), jnp.float32),
          pltpu.VMEM((_CHUNK,), jnp.float32),
          pltpu.VMEM((_CHUNK,), jnp.float32),
          pltpu.VMEM((_CHUNK,), jnp.float32),
          pltpu.VMEM((16,), jnp.float32),
          pltpu.VMEM((_NS * 16,), jnp.float32),
          pltpu.VMEM((_K * 3,), jnp.float32),
          pltpu.VMEM_SHARED((2 * _NS * 16,), jnp.float32),
      ],
  )
  def _fps_sc(x0_hbm, x1_hbm, x2_hbm, out_hbm, X0, X1, X2, D, rec, M, OUT,
              merge):
    w = lax.axis_index("s")
    base = w * _CHUNK
    iota = lax.iota(jnp.int32, 16)
    zeros_i = jnp.zeros((16,), jnp.int32)
    ones_i = jnp.full((16,), 1, jnp.int32)

    pltpu.sync_copy(x0_hbm.at[pl.ds(base, _CHUNK)], X0)
    pltpu.sync_copy(x1_hbm.at[pl.ds(base, _CHUNK)], X1)
    pltpu.sync_copy(x2_hbm.at[pl.ds(base, _CHUNK)], X2)

    pinf = jnp.full((16,), jnp.inf, jnp.float32)
    ninf = jnp.full((16,), -jnp.inf, jnp.float32)

    def initc(c, carry):
      gi = base + c * 16 + iota
      D[pl.ds(c * 16, 16)] = jnp.where(gi < _N, pinf, ninf)
      return carry

    lax.fori_loop(0, _NCH, initc, 0)

    def lane_sel(v0, v1, v2):
      return jnp.where(iota == 2, v0, jnp.where(iota == 3, v1, v2))

    two_i = jnp.full((16,), 2, jnp.int32)
    three_i = jnp.full((16,), 3, jnp.int32)
    four_i = jnp.full((16,), 4, jnp.int32)
    lane0 = iota == 0

    @pl.when(w == 0)
    def _():
      f0 = X0[pl.ds(0, 16)]
      f1 = X1[pl.ds(0, 16)]
      f2 = X2[pl.ds(0, 16)]
      plsc.store_scatter(rec, [two_i], f0, mask=lane0)
      plsc.store_scatter(rec, [three_i], f1, mask=lane0)
      plsc.store_scatter(rec, [four_i], f2, mask=lane0)
      pltpu.sync_copy(rec, merge.at[pl.ds(0, 16)])

    plsc.subcore_barrier()
    pltpu.sync_copy(merge.at[pl.ds(0, _NS * 16)], M)
    p0 = plsc.load_gather(M, [two_i])
    p1 = plsc.load_gather(M, [three_i])
    p2 = plsc.load_gather(M, [four_i])

    @pl.when(w == 0)
    def _():
      vals = jnp.where(iota == 0, p0, jnp.where(iota == 1, p1, p2))
      plsc.store_scatter(OUT, [iota], vals, mask=iota < 3)

    big_i = jnp.full((16,), 2**30, jnp.int32)
    big_f = jnp.full((16,), jnp.inf, jnp.float32)

    def itbody(i, p):
      p0, p1, p2 = p

      def upd(c, carry):
        m, moff = carry
        off = c * 16
        t0 = X0[pl.ds(off, 16)] - p0
        t1 = X1[pl.ds(off, 16)] - p1
        t2 = X2[pl.ds(off, 16)] - p2
        d = t0 * t0 + t1 * t1 + t2 * t2
        nd = jnp.minimum(D[pl.ds(off, 16)], d)
        D[pl.ds(off, 16)] = nd
        better = nd > m
        tie = (nd == m) & (off < moff)
        m = jnp.where(better, nd, m)
        moff = jnp.where(better | tie, off, moff)
        return m, moff

      m, moff = plsc.parallel_loop(0, _NCH, unroll=14,
                                   carry=(ninf, zeros_i))(upd)
      midx = moff + iota

      bm = jnp.max(m)
      lidx = jnp.min(jnp.where(m == bm, midx, big_i))
      lidx_v = lidx + zeros_i
      cp0 = plsc.load_gather(X0, [lidx_v])
      cp1 = plsc.load_gather(X1, [lidx_v])
      cp2 = plsc.load_gather(X2, [lidx_v])
      gidx_f = (base + lidx).astype(jnp.float32)

      rec_v = jnp.where(iota == 0, bm, jnp.where(iota == 1, gidx_f,
                                                 lane_sel(cp0, cp1, cp2)))
      rec[...] = rec_v
      par = (i % 2) * (_NS * 16)
      pltpu.sync_copy(rec, merge.at[pl.ds(par + w * 16, 16)])
      plsc.subcore_barrier()

      pltpu.sync_copy(merge.at[pl.ds(par, _NS * 16)], M)
      maxv = plsc.load_gather(M, [iota * 16])
      gidxv = plsc.load_gather(M, [iota * 16 + 1])
      gbm = jnp.max(maxv)
      bidx_f = jnp.min(jnp.where(maxv == gbm, gidxv, big_f))
      wrow = bidx_f.astype(jnp.int32) // _CHUNK
      np0 = plsc.load_gather(M, [wrow * 16 + two_i])
      np1 = plsc.load_gather(M, [wrow * 16 + three_i])
      np2 = plsc.load_gather(M, [wrow * 16 + four_i])

      @pl.when(w == 0)
      def _():
        vals = jnp.where(iota == 0, np0, jnp.where(iota == 1, np1, np2))
        plsc.store_scatter(OUT, [i * 3 + iota], vals, mask=iota < 3)

      return (np0, np1, np2)

    lax.fori_loop(1, _K, itbody, (p0, p1, p2))

    @pl.when(w == 0)
    def _():
      pltpu.sync_copy(OUT, out_hbm)

  return _fps_sc


def kernel(x):
  xT = jnp.zeros((3, _NPAD), jnp.float32).at[:, :_N].set(x.T)
  return _build_fps_sc()(xT[0], xT[1], xT[2]).reshape(_K, 3)


# speedup vs baseline: 1.0199x; 1.0199x over previous
import functools

import jax
import jax.numpy as jnp
from jax import lax
from jax.experimental import pallas as pl
from jax.experimental.pallas import tpu as pltpu
from jax.experimental.pallas import tpu_sc as plsc

_N = 50000
_K = 2048
_NS = 16
_CHUNK = 3136
_NCH = _CHUNK // 16
_NPAD = _NS * _CHUNK


@functools.cache
def _build_fps_sc():
  mesh = plsc.VectorSubcoreMesh(
      core_axis_name="c", subcore_axis_name="s", num_cores=1, num_subcores=_NS
  )

  @functools.partial(
      pl.kernel,
      out_type=jax.ShapeDtypeStruct((_K * 3,), jnp.float32),
      mesh=mesh,
      compiler_params=pltpu.CompilerParams(needs_layout_passes=False),
      scratch_types=[
          pltpu.VMEM((_CHUNK,), jnp.float32),
          pltpu.VMEM((_CHUNK,), jnp.float32),
          pltpu.VMEM((_CHUNK,), jnp.float32),
          pltpu.VMEM((_CHUNK,), jnp.float32),
          pltpu.VMEM((16,), jnp.float32),
          pltpu.VMEM((_NS * 16,), jnp.float32),
          pltpu.VMEM((_K * 3,), jnp.float32),
          pltpu.VMEM_SHARED((2 * _NS * 16,), jnp.float32),
      ],
  )
  def _fps_sc(x0_hbm, x1_hbm, x2_hbm, out_hbm, X0, X1, X2, D, rec, M, OUT,
              merge):
    w = lax.axis_index("s")
    base = w * _CHUNK
    iota = lax.iota(jnp.int32, 16)
    zeros_i = jnp.zeros((16,), jnp.int32)
    ones_i = jnp.full((16,), 1, jnp.int32)

    pltpu.sync_copy(x0_hbm.at[pl.ds(base, _CHUNK)], X0)
    pltpu.sync_copy(x1_hbm.at[pl.ds(base, _CHUNK)], X1)
    pltpu.sync_copy(x2_hbm.at[pl.ds(base, _CHUNK)], X2)

    pinf = jnp.full((16,), jnp.inf, jnp.float32)
    ninf = jnp.full((16,), -jnp.inf, jnp.float32)

    def initc(c, carry):
      gi = base + c * 16 + iota
      D[pl.ds(c * 16, 16)] = jnp.where(gi < _N, pinf, ninf)
      return carry

    lax.fori_loop(0, _NCH, initc, 0)

    def lane_sel(v0, v1, v2):
      return jnp.where(iota == 2, v0, jnp.where(iota == 3, v1, v2))

    two_i = jnp.full((16,), 2, jnp.int32)
    three_i = jnp.full((16,), 3, jnp.int32)
    four_i = jnp.full((16,), 4, jnp.int32)
    lane0 = iota == 0

    @pl.when(w == 0)
    def _():
      f0 = X0[pl.ds(0, 16)]
      f1 = X1[pl.ds(0, 16)]
      f2 = X2[pl.ds(0, 16)]
      plsc.store_scatter(rec, [two_i], f0, mask=lane0)
      plsc.store_scatter(rec, [three_i], f1, mask=lane0)
      plsc.store_scatter(rec, [four_i], f2, mask=lane0)
      pltpu.sync_copy(rec, merge.at[pl.ds(0, 16)])

    plsc.subcore_barrier()
    pltpu.sync_copy(merge.at[pl.ds(0, _NS * 16)], M)
    p0 = plsc.load_gather(M, [two_i])
    p1 = plsc.load_gather(M, [three_i])
    p2 = plsc.load_gather(M, [four_i])

    @pl.when(w == 0)
    def _():
      vals = jnp.where(iota == 0, p0, jnp.where(iota == 1, p1, p2))
      plsc.store_scatter(OUT, [iota], vals, mask=iota < 3)

    big_i = jnp.full((16,), 2**30, jnp.int32)
    big_f = jnp.full((16,), jnp.inf, jnp.float32)

    def itbody(i, p):
      p0, p1, p2 = p

      def upd(c, carry):
        m, midx = carry
        off = c * 16
        gi = off + iota
        t0 = X0[pl.ds(off, 16)] - p0
        t1 = X1[pl.ds(off, 16)] - p1
        t2 = X2[pl.ds(off, 16)] - p2
        d = t0 * t0 + t1 * t1 + t2 * t2
        nd = jnp.minimum(D[pl.ds(off, 16)], d)
        D[pl.ds(off, 16)] = nd
        better = nd > m
        tie = (nd == m) & (gi < midx)
        m = jnp.where(better, nd, m)
        midx = jnp.where(better | tie, gi, midx)
        return m, midx

      m, midx = plsc.parallel_loop(0, _NCH, unroll=14,
                                   carry=(ninf, zeros_i))(upd)

      bm = jnp.max(m)
      lidx = jnp.min(jnp.where(m == bm, midx, big_i))
      lidx_v = lidx + zeros_i
      cp0 = plsc.load_gather(X0, [lidx_v])
      cp1 = plsc.load_gather(X1, [lidx_v])
      cp2 = plsc.load_gather(X2, [lidx_v])
      gidx_f = (base + lidx).astype(jnp.float32)

      rec_v = jnp.where(iota == 0, bm, jnp.where(iota == 1, gidx_f,
                                                 lane_sel(cp0, cp1, cp2)))
      rec[...] = rec_v
      par = (i % 2) * (_NS * 16)
      pltpu.sync_copy(rec, merge.at[pl.ds(par + w * 16, 16)])
      plsc.subcore_barrier()

      pltpu.sync_copy(merge.at[pl.ds(par, _NS * 16)], M)
      maxv = plsc.load_gather(M, [iota * 16])
      gidxv = plsc.load_gather(M, [iota * 16 + 1])
      gbm = jnp.max(maxv)
      bidx_f = jnp.min(jnp.where(maxv == gbm, gidxv, big_f))
      wrow = plsc.all_reduce_ffs((maxv == gbm) & (gidxv == bidx_f))
      np0 = plsc.load_gather(M, [wrow * 16 + two_i])
      np1 = plsc.load_gather(M, [wrow * 16 + three_i])
      np2 = plsc.load_gather(M, [wrow * 16 + four_i])

      @pl.when(w == 0)
      def _():
        vals = jnp.where(iota == 0, np0, jnp.where(iota == 1, np1, np2))
        plsc.store_scatter(OUT, [i * 3 + iota], vals, mask=iota < 3)

      return (np0, np1, np2)

    lax.fori_loop(1, _K, itbody, (p0, p1, p2))

    @pl.when(w == 0)
    def _():
      pltpu.sync_copy(OUT, out_hbm)

  return _fps_sc


def kernel(x):
  xT = jnp.zeros((3, _NPAD), jnp.float32).at[:, :_N].set(x.T)
  return _build_fps_sc()(xT[0], xT[1], xT[2]).reshape(_K, 3)


# speedup vs baseline: 1.0567x; 1.0360x over previous
import functools

import jax
import jax.numpy as jnp
from jax import lax
from jax.experimental import pallas as pl
from jax.experimental.pallas import tpu as pltpu
from jax.experimental.pallas import tpu_sc as plsc

_N = 50000
_K = 2048
_NS = 16
_CHUNK = 3136
_NCH = _CHUNK // 16
_NPAD = _NS * _CHUNK


@functools.cache
def _build_fps_sc():
  mesh = plsc.VectorSubcoreMesh(
      core_axis_name="c", subcore_axis_name="s", num_cores=1, num_subcores=_NS
  )

  @functools.partial(
      pl.kernel,
      out_type=jax.ShapeDtypeStruct((_K * 3,), jnp.float32),
      mesh=mesh,
      compiler_params=pltpu.CompilerParams(needs_layout_passes=False),
      scratch_types=[
          pltpu.VMEM((_CHUNK,), jnp.float32),
          pltpu.VMEM((_CHUNK,), jnp.float32),
          pltpu.VMEM((_CHUNK,), jnp.float32),
          pltpu.VMEM((_CHUNK,), jnp.float32),
          pltpu.VMEM((16,), jnp.float32),
          pltpu.VMEM((_NS * 16,), jnp.float32),
          pltpu.VMEM((_K * 3,), jnp.float32),
          pltpu.VMEM_SHARED((2 * _NS * 16,), jnp.float32),
      ],
  )
  def _fps_sc(x0_hbm, x1_hbm, x2_hbm, out_hbm, X0, X1, X2, D, rec, M, OUT,
              merge):
    w = lax.axis_index("s")
    base = w * _CHUNK
    iota = lax.iota(jnp.int32, 16)
    zeros_i = jnp.zeros((16,), jnp.int32)

    pltpu.sync_copy(x0_hbm.at[pl.ds(base, _CHUNK)], X0)
    pltpu.sync_copy(x1_hbm.at[pl.ds(base, _CHUNK)], X1)
    pltpu.sync_copy(x2_hbm.at[pl.ds(base, _CHUNK)], X2)

    pinf = jnp.full((16,), jnp.inf, jnp.float32)
    ninf = jnp.full((16,), -jnp.inf, jnp.float32)

    def initc(c, carry):
      gi = base + c * 16 + iota
      D[pl.ds(c * 16, 16)] = jnp.where(gi < _N, pinf, ninf)
      return carry

    lax.fori_loop(0, _NCH, initc, 0)

    def lane_sel(v0, v1, v2):
      return jnp.where(iota == 2, v0, jnp.where(iota == 3, v1, v2))

    two_i = jnp.full((16,), 2, jnp.int32)
    three_i = jnp.full((16,), 3, jnp.int32)
    four_i = jnp.full((16,), 4, jnp.int32)
    lane0 = iota == 0

    @pl.when(w == 0)
    def _():
      f0 = X0[pl.ds(0, 16)]
      f1 = X1[pl.ds(0, 16)]
      f2 = X2[pl.ds(0, 16)]
      plsc.store_scatter(rec, [two_i], f0, mask=lane0)
      plsc.store_scatter(rec, [three_i], f1, mask=lane0)
      plsc.store_scatter(rec, [four_i], f2, mask=lane0)
      pltpu.sync_copy(rec, merge.at[pl.ds(0, 16)])

    plsc.subcore_barrier()
    pltpu.sync_copy(merge.at[pl.ds(0, _NS * 16)], M)
    p0 = plsc.load_gather(M, [two_i])
    p1 = plsc.load_gather(M, [three_i])
    p2 = plsc.load_gather(M, [four_i])

    @pl.when(w == 0)
    def _():
      vals = jnp.where(iota == 0, p0, jnp.where(iota == 1, p1, p2))
      plsc.store_scatter(OUT, [iota], vals, mask=iota < 3)

    big_i = jnp.full((16,), 2**30, jnp.int32)
    big_f = jnp.full((16,), jnp.inf, jnp.float32)

    def itbody(i, p):
      p0, p1, p2 = p

      def upd(c, carry):
        m, midx = carry
        off = c * 16
        gi = off + iota
        t0 = X0[pl.ds(off, 16)] - p0
        t1 = X1[pl.ds(off, 16)] - p1
        t2 = X2[pl.ds(off, 16)] - p2
        d = t0 * t0 + t1 * t1 + t2 * t2
        nd = jnp.minimum(D[pl.ds(off, 16)], d)
        D[pl.ds(off, 16)] = nd
        better = nd > m
        tie = (nd == m) & (gi < midx)
        m = jnp.where(better, nd, m)
        midx = jnp.where(better | tie, gi, midx)
        return m, midx

      m, midx = plsc.parallel_loop(0, _NCH, unroll=28,
                                   carry=(ninf, zeros_i))(upd)

      bm = jnp.max(m)
      lidx = jnp.min(jnp.where(m == bm, midx, big_i))
      lidx_v = lidx + zeros_i
      cp0 = plsc.load_gather(X0, [lidx_v])
      cp1 = plsc.load_gather(X1, [lidx_v])
      cp2 = plsc.load_gather(X2, [lidx_v])
      gidx_f = (base + lidx).astype(jnp.float32)

      rec_v = jnp.where(iota == 0, bm, jnp.where(iota == 1, gidx_f,
                                                 lane_sel(cp0, cp1, cp2)))
      rec[...] = rec_v
      par = (i % 2) * (_NS * 16)
      pltpu.sync_copy(rec, merge.at[pl.ds(par + w * 16, 16)])
      plsc.subcore_barrier()

      pltpu.sync_copy(merge.at[pl.ds(par, _NS * 16)], M)
      maxv = plsc.load_gather(M, [iota * 16])
      gidxv = plsc.load_gather(M, [iota * 16 + 1])
      gbm = jnp.max(maxv)
      bidx_f = jnp.min(jnp.where(maxv == gbm, gidxv, big_f))
      wrow = plsc.all_reduce_ffs((maxv == gbm) & (gidxv == bidx_f))
      np0 = plsc.load_gather(M, [wrow * 16 + two_i])
      np1 = plsc.load_gather(M, [wrow * 16 + three_i])
      np2 = plsc.load_gather(M, [wrow * 16 + four_i])

      @pl.when(w == 0)
      def _():
        vals = jnp.where(iota == 0, np0, jnp.where(iota == 1, np1, np2))
        plsc.store_scatter(OUT, [i * 3 + iota], vals, mask=iota < 3)

      return (np0, np1, np2)

    lax.fori_loop(1, _K, itbody, (p0, p1, p2))

    @pl.when(w == 0)
    def _():
      pltpu.sync_copy(OUT, out_hbm)

  return _fps_sc


def kernel(x):
  xT = jnp.zeros((3, _NPAD), jnp.float32).at[:, :_N].set(x.T)
  return _build_fps_sc()(xT[0], xT[1], xT[2]).reshape(_K, 3)


# speedup vs baseline: 1.0784x; 1.0206x over previous
import functools

import jax
import jax.numpy as jnp
from jax import lax
from jax.experimental import pallas as pl
from jax.experimental.pallas import tpu as pltpu
from jax.experimental.pallas import tpu_sc as plsc

_N = 50000
_K = 2048
_NS = 16
_CHUNK = 3136
_NCH = _CHUNK // 16
_NPAD = _NS * _CHUNK


@functools.cache
def _build_fps_sc():
  mesh = plsc.VectorSubcoreMesh(
      core_axis_name="c", subcore_axis_name="s", num_cores=1, num_subcores=_NS
  )

  @functools.partial(
      pl.kernel,
      out_type=jax.ShapeDtypeStruct((_K * 3,), jnp.float32),
      mesh=mesh,
      compiler_params=pltpu.CompilerParams(needs_layout_passes=False),
      scratch_types=[
          pltpu.VMEM((_CHUNK,), jnp.float32),
          pltpu.VMEM((_CHUNK,), jnp.float32),
          pltpu.VMEM((_CHUNK,), jnp.float32),
          pltpu.VMEM((_CHUNK,), jnp.float32),
          pltpu.VMEM((16,), jnp.float32),
          pltpu.VMEM((_NS * 16,), jnp.float32),
          pltpu.VMEM((_K * 3,), jnp.float32),
          pltpu.VMEM_SHARED((2 * _NS * 16,), jnp.float32),
      ],
  )
  def _fps_sc(x0_hbm, x1_hbm, x2_hbm, out_hbm, X0, X1, X2, D, rec, M, OUT,
              merge):
    w = lax.axis_index("s")
    base = w * _CHUNK
    iota = lax.iota(jnp.int32, 16)
    zeros_i = jnp.zeros((16,), jnp.int32)

    pltpu.sync_copy(x0_hbm.at[pl.ds(base, _CHUNK)], X0)
    pltpu.sync_copy(x1_hbm.at[pl.ds(base, _CHUNK)], X1)
    pltpu.sync_copy(x2_hbm.at[pl.ds(base, _CHUNK)], X2)

    pinf = jnp.full((16,), jnp.inf, jnp.float32)
    ninf = jnp.full((16,), -jnp.inf, jnp.float32)

    def initc(c, carry):
      gi = base + c * 16 + iota
      D[pl.ds(c * 16, 16)] = jnp.where(gi < _N, pinf, ninf)
      return carry

    lax.fori_loop(0, _NCH, initc, 0)

    def lane_sel(v0, v1, v2):
      return jnp.where(iota == 2, v0, jnp.where(iota == 3, v1, v2))

    two_i = jnp.full((16,), 2, jnp.int32)
    three_i = jnp.full((16,), 3, jnp.int32)
    four_i = jnp.full((16,), 4, jnp.int32)
    lane0 = iota == 0

    @pl.when(w == 0)
    def _():
      f0 = X0[pl.ds(0, 16)]
      f1 = X1[pl.ds(0, 16)]
      f2 = X2[pl.ds(0, 16)]
      plsc.store_scatter(rec, [two_i], f0, mask=lane0)
      plsc.store_scatter(rec, [three_i], f1, mask=lane0)
      plsc.store_scatter(rec, [four_i], f2, mask=lane0)
      pltpu.sync_copy(rec, merge.at[pl.ds(0, 16)])

    plsc.subcore_barrier()
    pltpu.sync_copy(merge.at[pl.ds(0, _NS * 16)], M)
    p0 = plsc.load_gather(M, [two_i])
    p1 = plsc.load_gather(M, [three_i])
    p2 = plsc.load_gather(M, [four_i])

    @pl.when(w == 0)
    def _():
      vals = jnp.where(iota == 0, p0, jnp.where(iota == 1, p1, p2))
      plsc.store_scatter(OUT, [iota], vals, mask=iota < 3)

    big_i = jnp.full((16,), 2**30, jnp.int32)
    big_f = jnp.full((16,), jnp.inf, jnp.float32)

    def itbody(i, p):
      p0, p1, p2 = p

      def upd(c, carry):
        m, midx = carry
        off = c * 16
        gi = off + iota
        t0 = X0[pl.ds(off, 16)] - p0
        t1 = X1[pl.ds(off, 16)] - p1
        t2 = X2[pl.ds(off, 16)] - p2
        d = t0 * t0 + t1 * t1 + t2 * t2
        nd = jnp.minimum(D[pl.ds(off, 16)], d)
        D[pl.ds(off, 16)] = nd
        better = nd > m
        tie = (nd == m) & (gi < midx)
        m = jnp.where(better, nd, m)
        midx = jnp.where(better | tie, gi, midx)
        return m, midx

      m, midx = plsc.parallel_loop(0, _NCH, unroll=49,
                                   carry=(ninf, zeros_i))(upd)

      bm = jnp.max(m)
      lidx = jnp.min(jnp.where(m == bm, midx, big_i))
      lidx_v = lidx + zeros_i
      cp0 = plsc.load_gather(X0, [lidx_v])
      cp1 = plsc.load_gather(X1, [lidx_v])
      cp2 = plsc.load_gather(X2, [lidx_v])
      gidx_f = (base + lidx).astype(jnp.float32)

      rec_v = jnp.where(iota == 0, bm, jnp.where(iota == 1, gidx_f,
                                                 lane_sel(cp0, cp1, cp2)))
      rec[...] = rec_v
      par = (i % 2) * (_NS * 16)
      pltpu.sync_copy(rec, merge.at[pl.ds(par + w * 16, 16)])
      plsc.subcore_barrier()

      pltpu.sync_copy(merge.at[pl.ds(par, _NS * 16)], M)
      maxv = plsc.load_gather(M, [iota * 16])
      gidxv = plsc.load_gather(M, [iota * 16 + 1])
      gbm = jnp.max(maxv)
      bidx_f = jnp.min(jnp.where(maxv == gbm, gidxv, big_f))
      wrow = plsc.all_reduce_ffs((maxv == gbm) & (gidxv == bidx_f))
      np0 = plsc.load_gather(M, [wrow * 16 + two_i])
      np1 = plsc.load_gather(M, [wrow * 16 + three_i])
      np2 = plsc.load_gather(M, [wrow * 16 + four_i])

      @pl.when(w == 0)
      def _():
        vals = jnp.where(iota == 0, np0, jnp.where(iota == 1, np1, np2))
        plsc.store_scatter(OUT, [i * 3 + iota], vals, mask=iota < 3)

      return (np0, np1, np2)

    lax.fori_loop(1, _K, itbody, (p0, p1, p2))

    @pl.when(w == 0)
    def _():
      pltpu.sync_copy(OUT, out_hbm)

  return _fps_sc


def kernel(x):
  xT = jnp.zeros((3, _NPAD), jnp.float32).at[:, :_N].set(x.T)
  return _build_fps_sc()(xT[0], xT[1], xT[2]).reshape(_K, 3)
